# baseline (speedup 1.0000x reference)
import functools

import jax
import jax.numpy as jnp
from jax import lax
from jax.experimental import pallas as pl
from jax.experimental.pallas import tpu as pltpu
from jax.experimental.pallas import tpu_sc as plsc

B = 16384
D = 64
WIDTH = 128
NE = 1000000

_HALF = 512000
_RC = 20480
_RNB = _HALF // _RC

_NC = 2
_NS = 16
_NW = _NC * _NS
_BPW = B // _NW
_CHUNK = 128
_NCHUNK = _BPW // _CHUNK

_NB = 16
_BLK = B // _NB


def _relayout_body(a_ref, b_ref, out_ref):
    out_ref[:, :D] = jnp.transpose(a_ref[...], (1, 0))
    out_ref[:, D:] = jnp.transpose(b_ref[...], (1, 0))


def _relayout(et):
    return pl.pallas_call(
        _relayout_body,
        grid=(_RNB,),
        in_specs=[
            pl.BlockSpec((D, _RC), lambda i: (0, i)),
            pl.BlockSpec(
                (D, _RC),
                lambda i: (0, jnp.minimum(_RNB + i, (NE - 1) // _RC))),
        ],
        out_specs=pl.BlockSpec((_RC, 2 * D), lambda i: (i, 0)),
        out_shape=jax.ShapeDtypeStruct((_HALF, 2 * D), jnp.float32),
    )(et, et)


def _sc_gather(e1r, relr, e2r, ent2, rel2):
    mesh = plsc.VectorSubcoreMesh(core_axis_name="c", subcore_axis_name="s")

    @functools.partial(
        pl.kernel,
        mesh=mesh,
        out_type=(
            jax.ShapeDtypeStruct((B, 2 * D), jnp.float32),
            jax.ShapeDtypeStruct((B, 2 * D), jnp.float32),
            jax.ShapeDtypeStruct((B, 2 * D), jnp.float32),
        ),
        scratch_types=[
            pltpu.VMEM((_NCHUNK, _CHUNK), jnp.int32),
            pltpu.VMEM((_NCHUNK, _CHUNK), jnp.int32),
            pltpu.VMEM((_NCHUNK, _CHUNK), jnp.int32),
            pltpu.VMEM((_BPW // 2, 2 * D), jnp.float32),
            pltpu.VMEM((_BPW // 2, 2 * D), jnp.float32),
            pltpu.VMEM((_BPW // 2, 2 * D), jnp.float32),
            pltpu.SemaphoreType.DMA,
        ],
        compiler_params=pltpu.CompilerParams(use_tc_tiling_on_sc=True),
    )
    def k(e1_hbm, rel_hbm, e2_hbm, ent_hbm, relemb_hbm,
          oh_hbm, or_hbm, ot_hbm,
          ih_v, ir_v, it_v, bh_v, br_v, bt_v, sem):
        wid = lax.axis_index("s") * _NC + lax.axis_index("c")
        row0 = wid * _NCHUNK
        base = wid * _BPW
        pltpu.sync_copy(e1_hbm.at[pl.ds(row0, _NCHUNK)], ih_v)
        pltpu.sync_copy(rel_hbm.at[pl.ds(row0, _NCHUNK)], ir_v)
        pltpu.sync_copy(e2_hbm.at[pl.ds(row0, _NCHUNK)], it_v)
        plane = ((ih_v, ent_hbm, oh_hbm, bh_v),
                 (ir_v, relemb_hbm, or_hbm, br_v),
                 (it_v, ent_hbm, ot_hbm, bt_v))
        for half in range(2):
            copies = []
            for j in (2 * half, 2 * half + 1):
                for idx_v, table, _, buf in plane:
                    copies.append(pltpu.async_copy(
                        table.at[idx_v.at[j]],
                        buf.at[pl.ds((j - 2 * half) * _CHUNK, _CHUNK)],
                        sem))
            for cp in copies:
                cp.wait()
            for _, _, out, buf in plane:
                pltpu.sync_copy(
                    buf, out.at[pl.ds(base + half * (_BPW // 2),
                                      _BPW // 2)])

    return k(e1r, relr, e2r, ent2, rel2)


def _mlp_body(h2_ref, r2_ref, t2_ref, ph_ref, pr_ref, pt_ref,
              w1h, w1r, w1t, b1, gamma, beta, w2, b2, out_ref,
              acc_ref, y_ref):
    phase = pl.program_id(0)
    i = pl.program_id(1)

    @pl.when(phase == 0)
    def _():
        h = jnp.where(ph_ref[...] > 0.5, h2_ref[:, D:], h2_ref[:, :D])
        r = jnp.where(pr_ref[...] > 0.5, r2_ref[:, D:], r2_ref[:, :D])
        t = jnp.where(pt_ref[...] > 0.5, t2_ref[:, D:], t2_ref[:, :D])
        y = jnp.dot(h, w1h[...], preferred_element_type=jnp.float32)
        y = y + jnp.dot(r, w1r[...], preferred_element_type=jnp.float32)
        y = y + jnp.dot(t, w1t[...], preferred_element_type=jnp.float32)
        y = y + b1[...]

        @pl.when(i == 0)
        def _():
            acc_ref[...] = jnp.zeros_like(acc_ref)

        acc_ref[0:1, :] += jnp.sum(y, axis=0, keepdims=True)
        acc_ref[1:2, :] += jnp.sum(y * y, axis=0, keepdims=True)
        y_ref[pl.ds(i * _BLK, _BLK), :] = y

    @pl.when(phase == 1)
    def _():
        mean = acc_ref[0:1, :] * (1.0 / B)
        var = acc_ref[1:2, :] * (1.0 / B) - mean * mean
        y = y_ref[pl.ds(i * _BLK, _BLK), :]
        z = (y - mean) * (gamma[...] * lax.rsqrt(var + 1e-5)) + beta[...]
        z = jnp.maximum(z, 0.0)
        o = jnp.dot(z, w2[...], preferred_element_type=jnp.float32) + b2[...]
        out_ref[...] = jax.nn.sigmoid(o)


def kernel(e1_idx, rel_idx, e2_idx, entity_emb, relation_emb,
           W1, b1, gamma, beta, W2, b2):
    e1_idx = e1_idx.astype(jnp.int32)
    rel_idx = rel_idx.astype(jnp.int32)
    e2_idx = e2_idx.astype(jnp.int32)

    ent2 = _relayout(entity_emb.T)
    rel2 = relation_emb.reshape(500, 2 * D)

    e1r = jnp.where(e1_idx >= _HALF, e1_idx - _HALF,
                    e1_idx).reshape(B // _CHUNK, _CHUNK)
    relr = (rel_idx >> 1).reshape(B // _CHUNK, _CHUNK)
    e2r = jnp.where(e2_idx >= _HALF, e2_idx - _HALF,
                    e2_idx).reshape(B // _CHUNK, _CHUNK)

    h2, r2, t2 = _sc_gather(e1r, relr, e2r, ent2, rel2)

    def par(mask):
        return jnp.broadcast_to(mask.astype(jnp.bfloat16).reshape(B, 1),
                                (B, D))

    blk = lambda p, i: (i * (1 - p), 0)
    whole = lambda p, i: (0, 0)
    out = pl.pallas_call(
        _mlp_body,
        grid=(2, _NB),
        in_specs=[
            pl.BlockSpec((_BLK, 2 * D), blk),
            pl.BlockSpec((_BLK, 2 * D), blk),
            pl.BlockSpec((_BLK, 2 * D), blk),
            pl.BlockSpec((_BLK, D), blk),
            pl.BlockSpec((_BLK, D), blk),
            pl.BlockSpec((_BLK, D), blk),
            pl.BlockSpec((D, WIDTH), whole),
            pl.BlockSpec((D, WIDTH), whole),
            pl.BlockSpec((D, WIDTH), whole),
            pl.BlockSpec((1, WIDTH), whole),
            pl.BlockSpec((1, WIDTH), whole),
            pl.BlockSpec((1, WIDTH), whole),
            pl.BlockSpec((WIDTH, 1), whole),
            pl.BlockSpec((1, 1), whole),
        ],
        out_specs=pl.BlockSpec((_BLK, 1), lambda p, i: (i, 0)),
        scratch_shapes=[pltpu.VMEM((2, WIDTH), jnp.float32),
                        pltpu.VMEM((B, WIDTH), jnp.float32)],
        out_shape=jax.ShapeDtypeStruct((B, 1), jnp.float32),
    )(h2, r2, t2, par(e1_idx >= _HALF), par(rel_idx & 1 > 0),
      par(e2_idx >= _HALF),
      W1[:D], W1[D:2 * D], W1[2 * D:],
      b1.reshape(1, WIDTH), gamma.reshape(1, WIDTH), beta.reshape(1, WIDTH),
      W2, b2.reshape(1, 1))
    return out


# speedup vs baseline: 1.0123x; 1.0123x over previous
import functools

import jax
import jax.numpy as jnp
from jax import lax
from jax.experimental import pallas as pl
from jax.experimental.pallas import tpu as pltpu
from jax.experimental.pallas import tpu_sc as plsc

B = 16384
D = 64
WIDTH = 128
NE = 1000000

_HALF = 507904
_RC = 16384
_RNB = _HALF // _RC

_NC = 2
_NS = 16
_NW = _NC * _NS
_BPW = B // _NW
_CHUNK = 128
_NCHUNK = _BPW // _CHUNK

_NB = 16
_BLK = B // _NB


def _relayout_body(a_ref, b_ref, out_ref):
    out_ref[:, :D] = jnp.transpose(a_ref[...], (1, 0))
    out_ref[:, D:] = jnp.transpose(b_ref[...], (1, 0))


def _relayout(et):
    return pl.pallas_call(
        _relayout_body,
        grid=(_RNB,),
        in_specs=[
            pl.BlockSpec((D, _RC), lambda i: (0, i)),
            pl.BlockSpec(
                (D, _RC),
                lambda i: (0, jnp.minimum(_RNB + i, (NE - 1) // _RC))),
        ],
        out_specs=pl.BlockSpec((_RC, 2 * D), lambda i: (i, 0)),
        out_shape=jax.ShapeDtypeStruct((_HALF, 2 * D), jnp.float32),
    )(et, et)


def _sc_gather(e1r, relr, e2r, ent2, rel2):
    mesh = plsc.VectorSubcoreMesh(core_axis_name="c", subcore_axis_name="s")

    @functools.partial(
        pl.kernel,
        mesh=mesh,
        out_type=(
            jax.ShapeDtypeStruct((B, 2 * D), jnp.float32),
            jax.ShapeDtypeStruct((B, 2 * D), jnp.float32),
            jax.ShapeDtypeStruct((B, 2 * D), jnp.float32),
        ),
        scratch_types=[
            pltpu.VMEM((_NCHUNK, _CHUNK), jnp.int32),
            pltpu.VMEM((_NCHUNK, _CHUNK), jnp.int32),
            pltpu.VMEM((_NCHUNK, _CHUNK), jnp.int32),
            pltpu.VMEM((_BPW // 2, 2 * D), jnp.float32),
            pltpu.VMEM((_BPW // 2, 2 * D), jnp.float32),
            pltpu.VMEM((_BPW // 2, 2 * D), jnp.float32),
            pltpu.SemaphoreType.DMA,
        ],
        compiler_params=pltpu.CompilerParams(use_tc_tiling_on_sc=True),
    )
    def k(e1_hbm, rel_hbm, e2_hbm, ent_hbm, relemb_hbm,
          oh_hbm, or_hbm, ot_hbm,
          ih_v, ir_v, it_v, bh_v, br_v, bt_v, sem):
        wid = lax.axis_index("s") * _NC + lax.axis_index("c")
        row0 = wid * _NCHUNK
        base = wid * _BPW
        pltpu.sync_copy(e1_hbm.at[pl.ds(row0, _NCHUNK)], ih_v)
        pltpu.sync_copy(rel_hbm.at[pl.ds(row0, _NCHUNK)], ir_v)
        pltpu.sync_copy(e2_hbm.at[pl.ds(row0, _NCHUNK)], it_v)
        plane = ((ih_v, ent_hbm, oh_hbm, bh_v),
                 (ir_v, relemb_hbm, or_hbm, br_v),
                 (it_v, ent_hbm, ot_hbm, bt_v))
        for half in range(2):
            copies = []
            for j in (2 * half, 2 * half + 1):
                for idx_v, table, _, buf in plane:
                    copies.append(pltpu.async_copy(
                        table.at[idx_v.at[j]],
                        buf.at[pl.ds((j - 2 * half) * _CHUNK, _CHUNK)],
                        sem))
            for cp in copies:
                cp.wait()
            for _, _, out, buf in plane:
                pltpu.sync_copy(
                    buf, out.at[pl.ds(base + half * (_BPW // 2),
                                      _BPW // 2)])

    return k(e1r, relr, e2r, ent2, rel2)


def _mlp_body(h2_ref, r2_ref, t2_ref, ph_ref, pr_ref, pt_ref,
              w1h, w1r, w1t, b1, gamma, beta, w2, b2, out_ref,
              acc_ref, y_ref):
    phase = pl.program_id(0)
    i = pl.program_id(1)

    @pl.when(phase == 0)
    def _():
        h = jnp.where(ph_ref[...] > 0.5, h2_ref[:, D:], h2_ref[:, :D])
        r = jnp.where(pr_ref[...] > 0.5, r2_ref[:, D:], r2_ref[:, :D])
        t = jnp.where(pt_ref[...] > 0.5, t2_ref[:, D:], t2_ref[:, :D])
        y = jnp.dot(h, w1h[...], preferred_element_type=jnp.float32)
        y = y + jnp.dot(r, w1r[...], preferred_element_type=jnp.float32)
        y = y + jnp.dot(t, w1t[...], preferred_element_type=jnp.float32)
        y = y + b1[...]

        @pl.when(i == 0)
        def _():
            acc_ref[...] = jnp.zeros_like(acc_ref)

        acc_ref[0:1, :] += jnp.sum(y, axis=0, keepdims=True)
        acc_ref[1:2, :] += jnp.sum(y * y, axis=0, keepdims=True)
        y_ref[pl.ds(i * _BLK, _BLK), :] = y

    @pl.when(phase == 1)
    def _():
        mean = acc_ref[0:1, :] * (1.0 / B)
        var = acc_ref[1:2, :] * (1.0 / B) - mean * mean
        y = y_ref[pl.ds(i * _BLK, _BLK), :]
        z = (y - mean) * (gamma[...] * lax.rsqrt(var + 1e-5)) + beta[...]
        z = jnp.maximum(z, 0.0)
        o = jnp.dot(z, w2[...], preferred_element_type=jnp.float32) + b2[...]
        out_ref[...] = jax.nn.sigmoid(o)


def kernel(e1_idx, rel_idx, e2_idx, entity_emb, relation_emb,
           W1, b1, gamma, beta, W2, b2):
    e1_idx = e1_idx.astype(jnp.int32)
    rel_idx = rel_idx.astype(jnp.int32)
    e2_idx = e2_idx.astype(jnp.int32)

    ent2 = _relayout(entity_emb.T)
    rel2 = relation_emb.reshape(500, 2 * D)

    e1r = jnp.where(e1_idx >= _HALF, e1_idx - _HALF,
                    e1_idx).reshape(B // _CHUNK, _CHUNK)
    relr = (rel_idx >> 1).reshape(B // _CHUNK, _CHUNK)
    e2r = jnp.where(e2_idx >= _HALF, e2_idx - _HALF,
                    e2_idx).reshape(B // _CHUNK, _CHUNK)

    h2, r2, t2 = _sc_gather(e1r, relr, e2r, ent2, rel2)

    def par(mask):
        return jnp.broadcast_to(mask.astype(jnp.bfloat16).reshape(B, 1),
                                (B, D))

    blk = lambda p, i: (i * (1 - p), 0)
    whole = lambda p, i: (0, 0)
    out = pl.pallas_call(
        _mlp_body,
        grid=(2, _NB),
        in_specs=[
            pl.BlockSpec((_BLK, 2 * D), blk),
            pl.BlockSpec((_BLK, 2 * D), blk),
            pl.BlockSpec((_BLK, 2 * D), blk),
            pl.BlockSpec((_BLK, D), blk),
            pl.BlockSpec((_BLK, D), blk),
            pl.BlockSpec((_BLK, D), blk),
            pl.BlockSpec((D, WIDTH), whole),
            pl.BlockSpec((D, WIDTH), whole),
            pl.BlockSpec((D, WIDTH), whole),
            pl.BlockSpec((1, WIDTH), whole),
            pl.BlockSpec((1, WIDTH), whole),
            pl.BlockSpec((1, WIDTH), whole),
            pl.BlockSpec((WIDTH, 1), whole),
            pl.BlockSpec((1, 1), whole),
        ],
        out_specs=pl.BlockSpec((_BLK, 1), lambda p, i: (i, 0)),
        scratch_shapes=[pltpu.VMEM((2, WIDTH), jnp.float32),
                        pltpu.VMEM((B, WIDTH), jnp.float32)],
        out_shape=jax.ShapeDtypeStruct((B, 1), jnp.float32),
    )(h2, r2, t2, par(e1_idx >= _HALF), par(rel_idx & 1 > 0),
      par(e2_idx >= _HALF),
      W1[:D], W1[D:2 * D], W1[2 * D:],
      b1.reshape(1, WIDTH), gamma.reshape(1, WIDTH), beta.reshape(1, WIDTH),
      W2, b2.reshape(1, 1))
    return out
